# baseline (speedup 1.0000x reference)
import functools

import jax
import jax.numpy as jnp
from jax.experimental import pallas as pl


def _body(a_ref, t_ref, o_ref):
    blk, n, k = a_ref.shape
    d = t_ref.shape[-1]
    a = a_ref[...].reshape(blk * n, k)
    o = jnp.dot(a, t_ref[...], preferred_element_type=jnp.float32)
    o_ref[...] = o.reshape(blk, n, d)


def kernel(alignments, alignment_embeds):
    B, N, K = alignments.shape
    D = alignment_embeds.shape[-1]
    BBLK = 32
    out = pl.pallas_call(
        _body,
        grid=(B // BBLK,),
        in_specs=[
            pl.BlockSpec((BBLK, N, K), lambda i: (i, 0, 0)),
            pl.BlockSpec((K, D), lambda i: (0, 0)),
        ],
        out_specs=pl.BlockSpec((BBLK, N, D), lambda i: (i, 0, 0)),
        out_shape=jax.ShapeDtypeStruct((B, N, D), jnp.float32),
    )(alignments, alignment_embeds)
    return out


# speedup vs baseline: 2.8026x; 2.8026x over previous
import functools

import jax
import jax.numpy as jnp
from jax.experimental import pallas as pl


def _body(a_ref, t_ref, o_ref):
    o_ref[...] = jax.lax.dot_general(
        a_ref[...], t_ref[...],
        dimension_numbers=(((0,), (0,)), ((), ())),
        preferred_element_type=jnp.float32,
    )


def kernel(alignments, alignment_embeds):
    B, N, K = alignments.shape
    D = alignment_embeds.shape[-1]
    BBLK = 64
    at = jnp.transpose(alignments, (2, 0, 1))
    out = pl.pallas_call(
        _body,
        grid=(B // BBLK,),
        in_specs=[
            pl.BlockSpec((K, BBLK, N), lambda i: (0, i, 0)),
            pl.BlockSpec((K, D), lambda i: (0, 0)),
        ],
        out_specs=pl.BlockSpec((BBLK, N, D), lambda i: (i, 0, 0)),
        out_shape=jax.ShapeDtypeStruct((B, N, D), jnp.float32),
    )(at, alignment_embeds)
    return out


# speedup vs baseline: 3.0248x; 1.0793x over previous
import functools

import jax
import jax.numpy as jnp
from jax.experimental import pallas as pl


def _body(a_ref, t_ref, o_ref):
    o_ref[...] = jax.lax.dot_general(
        a_ref[...].astype(jnp.bfloat16), t_ref[...].astype(jnp.bfloat16),
        dimension_numbers=(((0,), (0,)), ((), ())),
        preferred_element_type=jnp.float32,
    )


def kernel(alignments, alignment_embeds):
    B, N, K = alignments.shape
    D = alignment_embeds.shape[-1]
    BBLK = 64
    at = jnp.transpose(alignments, (2, 0, 1))
    out = pl.pallas_call(
        _body,
        grid=(B // BBLK,),
        in_specs=[
            pl.BlockSpec((K, BBLK, N), lambda i: (0, i, 0)),
            pl.BlockSpec((K, D), lambda i: (0, 0)),
        ],
        out_specs=pl.BlockSpec((BBLK, N, D), lambda i: (i, 0, 0)),
        out_shape=jax.ShapeDtypeStruct((B, N, D), jnp.float32),
    )(at, alignment_embeds)
    return out
